# baseline (speedup 1.0000x reference)
import functools

import jax
import jax.numpy as jnp
import numpy as np
from jax import lax
from jax.experimental import pallas as pl
from jax.experimental.pallas import tpu as pltpu
from jax.experimental.pallas import tpu_sc as plsc

_L = 16
_NC = 2
_NS = 16
_NW = _NC * _NS


def _sinusoidal_pe_np(seq_len: int, d_model: int) -> np.ndarray:
    pos = np.arange(seq_len, dtype=np.float32)[:, None]
    i = np.arange(0, d_model, 2, dtype=np.float32)
    div = np.exp(-(np.log(10000.0)) * i / d_model)
    pe = np.zeros((seq_len, d_model), dtype=np.float32)
    pe[:, 0::2] = np.sin(pos * div)
    pe[:, 1::2] = np.cos(pos * div)
    return pe


@functools.partial(jax.jit, static_argnames=("batch", "seq", "d_model"))
def _embed(x_flat, table, pe, *, batch, seq, d_model):
    P = seq // _NW
    C = 32
    n_h = P // C
    nchunks = batch * n_h

    NB = 3
    mesh = plsc.VectorSubcoreMesh(core_axis_name="c", subcore_axis_name="s")

    @functools.partial(
        pl.kernel,
        out_type=jax.ShapeDtypeStruct((batch * seq, d_model), jnp.float32),
        mesh=mesh,
        scratch_types=[
            pltpu.VMEM((P, d_model), jnp.float32),
            [pltpu.VMEM((C, d_model), jnp.float32)] * NB,
            pltpu.VMEM((batch * P,), jnp.int32),
            [pltpu.SemaphoreType.DMA] * NB,
            [pltpu.SemaphoreType.DMA] * NB,
            pltpu.SemaphoreType.DMA,
            pltpu.SemaphoreType.DMA,
        ],
    )
    def body(x_hbm, table_hbm, pe_hbm, out_hbm,
             pe_v, toks, idx_all, gsems, ssems, pesem, isem):
        wid = lax.axis_index("s") * _NC + lax.axis_index("c")
        pos0 = wid * P
        pe_dma = pltpu.async_copy(pe_hbm.at[pl.ds(pos0, P)], pe_v, pesem)
        idx_dmas = [
            pltpu.async_copy(x_hbm.at[pl.ds(b * seq + pos0, P)],
                             idx_all.at[pl.ds(b * P, P)], isem)
            for b in range(batch)
        ]
        for d in idx_dmas:
            d.wait()

        def offs(c):
            b, h = divmod(c, n_h)
            return b * P + h * C, b * seq + pos0 + h * C, h

        gathers, stores = {}, {}

        def issue_gather(c):
            pb = c % NB
            ioff, _, _ = offs(c)
            gathers[c] = pltpu.async_copy(
                table_hbm.at[idx_all.at[pl.ds(ioff, C)]], toks[pb], gsems[pb])

        issue_gather(0)
        if nchunks > 1:
            issue_gather(1)

        for c in range(nchunks):
            pb = c % NB
            if c + 2 < nchunks:
                if c >= 1:
                    stores[c - 1].wait()
                issue_gather(c + 2)
            gathers[c].wait()
            if c == 0:
                pe_dma.wait()
            _, obase, h = offs(c)

            @pl.loop(0, C, step=2)
            def _(r):
                for rr in range(2):
                    for col in range(0, d_model, _L):
                        plsc.addupdate(toks[pb].at[r + rr, pl.ds(col, _L)],
                                       pe_v[h * C + r + rr, pl.ds(col, _L)])

            stores[c] = pltpu.async_copy(
                toks[pb], out_hbm.at[pl.ds(obase, C)], ssems[pb])

        for c in range(max(0, nchunks - 3), nchunks):
            stores[c].wait()

    return body(x_flat, table, pe)


def kernel(x, token_table):
    batch, seq = x.shape
    d_model = token_table.shape[1]
    pe = jnp.asarray(_sinusoidal_pe_np(seq, d_model))
    x_flat = x.reshape(batch * seq).astype(jnp.int32)
    out = _embed(x_flat, token_table, pe,
                 batch=batch, seq=seq, d_model=d_model)
    return out.reshape(batch, seq, d_model)


# speedup vs baseline: 1.0770x; 1.0770x over previous
import functools

import jax
import jax.numpy as jnp
import numpy as np
from jax import lax
from jax.experimental import pallas as pl
from jax.experimental.pallas import tpu as pltpu
from jax.experimental.pallas import tpu_sc as plsc

_L = 16
_NC = 2
_NS = 16
_NW = _NC * _NS


def _sinusoidal_pe_np(seq_len: int, d_model: int) -> np.ndarray:
    pos = np.arange(seq_len, dtype=np.float32)[:, None]
    i = np.arange(0, d_model, 2, dtype=np.float32)
    div = np.exp(-(np.log(10000.0)) * i / d_model)
    pe = np.zeros((seq_len, d_model), dtype=np.float32)
    pe[:, 0::2] = np.sin(pos * div)
    pe[:, 1::2] = np.cos(pos * div)
    return pe


@functools.partial(jax.jit, static_argnames=("batch", "seq", "d_model"))
def _embed(x_flat, table, pe, *, batch, seq, d_model):
    P = seq // _NW
    C = 16
    n_h = P // C
    nchunks = batch * n_h

    NB = 4
    mesh = plsc.VectorSubcoreMesh(core_axis_name="c", subcore_axis_name="s")

    @functools.partial(
        pl.kernel,
        out_type=jax.ShapeDtypeStruct((batch * seq, d_model), jnp.float32),
        mesh=mesh,
        scratch_types=[
            pltpu.VMEM((P, d_model), jnp.float32),
            [pltpu.VMEM((C, d_model), jnp.float32)] * NB,
            pltpu.VMEM((batch * P,), jnp.int32),
            [pltpu.SemaphoreType.DMA] * NB,
            [pltpu.SemaphoreType.DMA] * NB,
            pltpu.SemaphoreType.DMA,
            pltpu.SemaphoreType.DMA,
        ],
    )
    def body(x_hbm, table_hbm, pe_hbm, out_hbm,
             pe_v, toks, idx_all, gsems, ssems, pesem, isem):
        wid = lax.axis_index("s") * _NC + lax.axis_index("c")
        pos0 = wid * P
        pe_dma = pltpu.async_copy(pe_hbm.at[pl.ds(pos0, P)], pe_v, pesem)
        idx_dmas = [
            pltpu.async_copy(x_hbm.at[pl.ds(b * seq + pos0, P)],
                             idx_all.at[pl.ds(b * P, P)], isem)
            for b in range(batch)
        ]
        for d in idx_dmas:
            d.wait()

        def offs(c):
            b, h = divmod(c, n_h)
            return b * P + h * C, b * seq + pos0 + h * C, h

        gathers, stores = {}, {}

        def issue_gather(c):
            pb = c % NB
            ioff, _, _ = offs(c)
            gathers[c] = pltpu.async_copy(
                table_hbm.at[idx_all.at[pl.ds(ioff, C)]], toks[pb], gsems[pb])

        issue_gather(0)
        if nchunks > 1:
            issue_gather(1)

        for c in range(nchunks):
            pb = c % NB
            if c + 2 < nchunks:
                if c >= 2:
                    stores[c - 2].wait()
                issue_gather(c + 2)
            gathers[c].wait()
            if c == 0:
                pe_dma.wait()
            _, obase, h = offs(c)

            @pl.loop(0, C)
            def _(r):
                for col in range(0, d_model, _L):
                    plsc.addupdate(toks[pb].at[r, pl.ds(col, _L)],
                                   pe_v[h * C + r, pl.ds(col, _L)])

            stores[c] = pltpu.async_copy(
                toks[pb], out_hbm.at[pl.ds(obase, C)], ssems[pb])

        for c in range(max(0, nchunks - 4), nchunks):
            stores[c].wait()

    return body(x_flat, table, pe)


def kernel(x, token_table):
    batch, seq = x.shape
    d_model = token_table.shape[1]
    pe = jnp.asarray(_sinusoidal_pe_np(seq, d_model))
    x_flat = x.reshape(batch * seq).astype(jnp.int32)
    out = _embed(x_flat, token_table, pe,
                 batch=batch, seq=seq, d_model=d_model)
    return out.reshape(batch, seq, d_model)


# speedup vs baseline: 1.1340x; 1.0529x over previous
import functools

import jax
import jax.numpy as jnp
import numpy as np
from jax import lax
from jax.experimental import pallas as pl
from jax.experimental.pallas import tpu as pltpu
from jax.experimental.pallas import tpu_sc as plsc

_L = 16
_NC = 2
_NS = 16
_NW = _NC * _NS


def _sinusoidal_pe_np(seq_len: int, d_model: int) -> np.ndarray:
    pos = np.arange(seq_len, dtype=np.float32)[:, None]
    i = np.arange(0, d_model, 2, dtype=np.float32)
    div = np.exp(-(np.log(10000.0)) * i / d_model)
    pe = np.zeros((seq_len, d_model), dtype=np.float32)
    pe[:, 0::2] = np.sin(pos * div)
    pe[:, 1::2] = np.cos(pos * div)
    return pe


@functools.partial(jax.jit, static_argnames=("batch", "seq", "d_model"))
def _embed(x_perm, oidx, table, pe, *, batch, seq, d_model):
    P = seq // _NW
    G = 4
    C = G * batch
    nchunks = P // G
    NB = 4

    mesh = plsc.VectorSubcoreMesh(core_axis_name="c", subcore_axis_name="s")

    @functools.partial(
        pl.kernel,
        out_type=jax.ShapeDtypeStruct((batch * seq, d_model), jnp.float32),
        mesh=mesh,
        scratch_types=[
            pltpu.VMEM((P, d_model), jnp.float32),
            [pltpu.VMEM((C, d_model), jnp.float32)] * NB,
            pltpu.VMEM((nchunks, C), jnp.int32),
            pltpu.VMEM((nchunks, C), jnp.int32),
            [pltpu.SemaphoreType.DMA] * NB,
            [pltpu.SemaphoreType.DMA] * NB,
            pltpu.SemaphoreType.DMA,
            pltpu.SemaphoreType.DMA,
        ],
    )
    def body(x_hbm, oidx_hbm, table_hbm, pe_hbm, out_hbm,
             pe_v, toks, idx_v, oidx_v, gsems, ssems, pesem, isem):
        wid = lax.axis_index("s") * _NC + lax.axis_index("c")
        pos0 = wid * P
        pe_dma = pltpu.async_copy(pe_hbm.at[pl.ds(pos0, P)], pe_v, pesem)
        i_dma = pltpu.async_copy(x_hbm.at[wid], idx_v, isem)
        o_dma = pltpu.async_copy(oidx_hbm.at[wid], oidx_v, isem)
        i_dma.wait()
        o_dma.wait()

        gathers, stores = {}, {}

        def issue_gather(c):
            pb = c % NB
            gathers[c] = pltpu.async_copy(
                table_hbm.at[idx_v.at[c]], toks[pb], gsems[pb])

        issue_gather(0)
        issue_gather(1)

        for c in range(nchunks):
            pb = c % NB
            if c + 2 < nchunks:
                if c >= 2:
                    stores[c - 2].wait()
                issue_gather(c + 2)
            gathers[c].wait()
            if c == 0:
                pe_dma.wait()

            @pl.loop(0, G)
            def _(p):
                for col in range(0, d_model, _L):
                    pe_val = pe_v[c * G + p, pl.ds(col, _L)]
                    for b in range(batch):
                        plsc.addupdate(toks[pb].at[b * G + p, pl.ds(col, _L)],
                                       pe_val)

            stores[c] = pltpu.async_copy(
                toks[pb], out_hbm.at[oidx_v.at[c]], ssems[pb])

        for c in range(max(0, nchunks - 4), nchunks):
            stores[c].wait()

    return body(x_perm, oidx, table, pe)


def kernel(x, token_table):
    batch, seq = x.shape
    d_model = token_table.shape[1]
    P = seq // _NW
    G = 4
    C = G * batch
    nchunks = P // G

    pe = jnp.asarray(_sinusoidal_pe_np(seq, d_model))
    x_perm = (x.astype(jnp.int32)
               .reshape(batch, _NW, nchunks, G)
               .transpose(1, 2, 0, 3)
               .reshape(_NW, nchunks, C))
    b_i, w_i, k_i, p_i = np.meshgrid(
        np.arange(batch), np.arange(_NW), np.arange(nchunks), np.arange(G),
        indexing="ij")
    orow = (b_i * seq + w_i * P + k_i * G + p_i).astype(np.int32)
    oidx = jnp.asarray(
        orow.transpose(1, 2, 0, 3).reshape(_NW, nchunks, C))

    out = _embed(x_perm, oidx, token_table, pe,
                 batch=batch, seq=seq, d_model=d_model)
    return out.reshape(batch, seq, d_model)


# speedup vs baseline: 1.2359x; 1.0899x over previous
import functools

import jax
import jax.numpy as jnp
import numpy as np
from jax import lax
from jax.experimental import pallas as pl
from jax.experimental.pallas import tpu as pltpu
from jax.experimental.pallas import tpu_sc as plsc

_L = 16
_NC = 2
_NS = 16
_NW = _NC * _NS


def _sinusoidal_pe_np(seq_len: int, d_model: int) -> np.ndarray:
    pos = np.arange(seq_len, dtype=np.float32)[:, None]
    i = np.arange(0, d_model, 2, dtype=np.float32)
    div = np.exp(-(np.log(10000.0)) * i / d_model)
    pe = np.zeros((seq_len, d_model), dtype=np.float32)
    pe[:, 0::2] = np.sin(pos * div)
    pe[:, 1::2] = np.cos(pos * div)
    return pe


@functools.partial(jax.jit, static_argnames=("batch", "seq", "d_model"))
def _embed(x_perm, oidx, table, pe, *, batch, seq, d_model):
    P = seq // _NW
    G = 8
    C = G * batch
    nchunks = P // G
    NB = 3

    mesh = plsc.VectorSubcoreMesh(core_axis_name="c", subcore_axis_name="s")

    @functools.partial(
        pl.kernel,
        out_type=jax.ShapeDtypeStruct((batch * seq, d_model), jnp.float32),
        mesh=mesh,
        scratch_types=[
            pltpu.VMEM((P, d_model), jnp.float32),
            [pltpu.VMEM((C, d_model), jnp.float32)] * NB,
            pltpu.VMEM((nchunks, C), jnp.int32),
            pltpu.VMEM((nchunks, C), jnp.int32),
            [pltpu.SemaphoreType.DMA] * NB,
            [pltpu.SemaphoreType.DMA] * NB,
            pltpu.SemaphoreType.DMA,
            pltpu.SemaphoreType.DMA,
        ],
    )
    def body(x_hbm, oidx_hbm, table_hbm, pe_hbm, out_hbm,
             pe_v, toks, idx_v, oidx_v, gsems, ssems, pesem, isem):
        wid = lax.axis_index("s") * _NC + lax.axis_index("c")
        pos0 = wid * P
        pe_dma = pltpu.async_copy(pe_hbm.at[pl.ds(pos0, P)], pe_v, pesem)
        i_dma = pltpu.async_copy(x_hbm.at[wid], idx_v, isem)
        o_dma = pltpu.async_copy(oidx_hbm.at[wid], oidx_v, isem)
        i_dma.wait()
        o_dma.wait()

        gathers, stores = {}, {}

        def issue_gather(c):
            pb = c % NB
            gathers[c] = pltpu.async_copy(
                table_hbm.at[idx_v.at[c]], toks[pb], gsems[pb])

        issue_gather(0)

        for c in range(nchunks):
            pb = c % NB
            if c + 1 < nchunks:
                if c >= 2:
                    stores[c - 2].wait()
                issue_gather(c + 1)
            gathers[c].wait()
            if c == 0:
                pe_dma.wait()

            @pl.loop(0, G)
            def _(p):
                for col in range(0, d_model, _L):
                    pe_val = pe_v[c * G + p, pl.ds(col, _L)]
                    for b in range(batch):
                        plsc.addupdate(toks[pb].at[b * G + p, pl.ds(col, _L)],
                                       pe_val)

            stores[c] = pltpu.async_copy(
                toks[pb], out_hbm.at[oidx_v.at[c]], ssems[pb])

        for c in range(max(0, nchunks - 3), nchunks):
            stores[c].wait()

    return body(x_perm, oidx, table, pe)


def kernel(x, token_table):
    batch, seq = x.shape
    d_model = token_table.shape[1]
    P = seq // _NW
    G = 8
    C = G * batch
    nchunks = P // G

    pe = jnp.asarray(_sinusoidal_pe_np(seq, d_model))
    x_perm = (x.astype(jnp.int32)
               .reshape(batch, _NW, nchunks, G)
               .transpose(1, 2, 0, 3)
               .reshape(_NW, nchunks, C))
    b_i, w_i, k_i, p_i = np.meshgrid(
        np.arange(batch), np.arange(_NW), np.arange(nchunks), np.arange(G),
        indexing="ij")
    orow = (b_i * seq + w_i * P + k_i * G + p_i).astype(np.int32)
    oidx = jnp.asarray(
        orow.transpose(1, 2, 0, 3).reshape(_NW, nchunks, C))

    out = _embed(x_perm, oidx, token_table, pe,
                 batch=batch, seq=seq, d_model=d_model)
    return out.reshape(batch, seq, d_model)
